# baseline (speedup 1.0000x reference)
import math

import jax
import jax.numpy as jnp
from jax.experimental import pallas as pl
from jax.experimental.pallas import tpu as pltpu

_B, _H, _L, _DK = 4, 32, 4096, 64
_LK = 4096
_NC = 3
_LOG_L = 8
_LOG_LK = 8
_HB = 4


def _main_kernel(q_ref, k_ref, v_ref, wskT_ref, bsk_ref, wsvT_ref, bsv_ref,
                 wsqT_ref, bsq_ref, wpkT_ref, wpqT_ref,
                 bpq_ref, bpk_ref, wqp_ref, bqp_ref, wkp_ref, bkp_ref,
                 vm_ref, loss_ref, zq_acc, zk_acc):
    b = pl.program_id(0)
    h = pl.program_id(1)

    @pl.when(jnp.logical_and(b == 0, h == 0))
    def _init():
        zq_acc[...] = jnp.zeros_like(zq_acc)
        zk_acc[...] = jnp.zeros_like(zk_acc)

    zpad = jnp.zeros((1, 128 - _NC), jnp.float32)
    cq_parts = []
    ck_parts = []
    for hh in range(_HB):
        q = q_ref[0, hh]
        k = k_ref[0, hh]
        v = v_ref[0, hh]

        ks = jnp.dot(k, wskT_ref[...], preferred_element_type=jnp.float32) + bsk_ref[...]
        qs = jnp.dot(q, wsqT_ref[...], preferred_element_type=jnp.float32) + bsq_ref[...]
        vs = jnp.dot(v, wsvT_ref[...], preferred_element_type=jnp.float32) + bsv_ref[...]

        vm = jnp.mean(vs, axis=1, keepdims=True)
        vm_ref[0, hh] = jnp.broadcast_to(vm, (_DK, 128))

        ck_parts.append(jnp.concatenate(
            [jnp.sum(ks * wpkT_ref[c, hh], keepdims=True) for c in range(_NC)]
            + [zpad], axis=1))
        cq_parts.append(jnp.concatenate(
            [jnp.sum(qs * wpqT_ref[c, hh], keepdims=True) for c in range(_NC)]
            + [zpad], axis=1))

    onehot = (jax.lax.broadcasted_iota(jnp.int32, (_B, 1), 0) == b
              ).astype(jnp.float32)
    zq_acc[...] += onehot * sum(cq_parts)
    zk_acc[...] += onehot * sum(ck_parts)

    @pl.when(jnp.logical_and(b == _B - 1, h == _H // _HB - 1))
    def _finish():
        zq = zq_acc[...][:, :_NC]
        zk = zk_acc[...][:, :_NC]
        cqp = jnp.maximum(zq + bpq_ref[...], 0.0)
        ckp = jnp.maximum(zk + bpk_ref[...], 0.0)
        logit_q = jnp.dot(cqp, wqp_ref[...].T,
                          preferred_element_type=jnp.float32) + bqp_ref[...]
        logit_k = jnp.dot(ckp, wkp_ref[...].T,
                          preferred_element_type=jnp.float32) + bkp_ref[...]
        cluster_q = jax.nn.softmax(logit_q, axis=-1)
        cluster_k = jax.nn.softmax(logit_k, axis=-1)
        mu = jnp.mean(cluster_q, axis=0, keepdims=True)
        mk = jnp.mean(cluster_k, axis=0, keepdims=True)
        var = jnp.sum((cluster_k - mk) ** 2, axis=0, keepdims=True) / (_B - 1)
        sigma = jax.nn.softplus(jnp.sqrt(var))
        ll = (-0.5 * ((cluster_k - mu) / sigma) ** 2 - jnp.log(sigma)
              - 0.5 * math.log(2.0 * math.pi))
        lsm = jax.nn.log_softmax(cluster_q, axis=-1)
        ce_terms = jnp.sum(-cluster_q * lsm, axis=-1, keepdims=True)
        loss_ref[...] = (jnp.mean(ce_terms, axis=0, keepdims=True)
                         - jnp.mean(ll, keepdims=True))


def _bcast_kernel(vm_ref, ctx_ref):
    for hh in range(_HB):
        ctx_ref[0, hh] = jnp.broadcast_to(vm_ref[0, hh][:, :1], (_DK, _L))


def kernel(Q, K, V, W_sk, b_sk, W_sv, b_sv, W_sq, b_sq, W_pk, b_pk,
           W_pq, b_pq, W_qp, b_qp, W_kp, b_kp):
    Qt = jnp.swapaxes(Q, 2, 3)
    Kt = jnp.swapaxes(K, 2, 3)
    Vt = jnp.swapaxes(V, 2, 3)

    qkv_spec = pl.BlockSpec((1, _HB, _DK, _L), lambda b, h: (b, h, 0, 0))
    w_spec = pl.BlockSpec((_LK, _LOG_LK), lambda b, h: (0, 0))
    bias_spec = pl.BlockSpec((1, _LOG_LK), lambda b, h: (0, 0))
    wp_spec = pl.BlockSpec((_NC, _HB, _DK, _LOG_LK), lambda b, h: (0, h, 0, 0))
    small_spec = lambda r, c: pl.BlockSpec((r, c), lambda b, h: (0, 0))

    vm_all, loss = pl.pallas_call(
        _main_kernel,
        grid=(_B, _H // _HB),
        in_specs=[qkv_spec, qkv_spec, qkv_spec,
                  w_spec, bias_spec, w_spec, bias_spec, w_spec, bias_spec,
                  wp_spec, wp_spec,
                  small_spec(1, _NC), small_spec(1, _NC),
                  small_spec(_NC, _NC), small_spec(1, _NC),
                  small_spec(_NC, _NC), small_spec(1, _NC)],
        out_specs=[pl.BlockSpec((1, _HB, _DK, 128), lambda b, h: (b, h, 0, 0)),
                   pl.BlockSpec((1, 1), lambda b, h: (0, 0))],
        out_shape=[
            jax.ShapeDtypeStruct((_B, _H, _DK, 128), jnp.float32),
            jax.ShapeDtypeStruct((1, 1), jnp.float32),
        ],
        scratch_shapes=[pltpu.VMEM((_B, 128), jnp.float32),
                        pltpu.VMEM((_B, 128), jnp.float32)],
        compiler_params=pltpu.CompilerParams(
            dimension_semantics=("arbitrary", "arbitrary"),
            vmem_limit_bytes=100 * 1024 * 1024),
    )(Qt, Kt, Vt,
      W_sk.T, b_sk.reshape(1, _LOG_LK), W_sv.T, b_sv.reshape(1, _LOG_LK),
      W_sq.T, b_sq.reshape(1, _LOG_L),
      W_pk.reshape(_NC, _H, _LOG_LK, _DK).transpose(0, 1, 3, 2),
      W_pq.reshape(_NC, _H, _LOG_L, _DK).transpose(0, 1, 3, 2),
      b_pq.reshape(1, _NC), b_pk.reshape(1, _NC),
      W_qp, b_qp.reshape(1, _NC), W_kp, b_kp.reshape(1, _NC))

    ctx_t = pl.pallas_call(
        _bcast_kernel,
        grid=(_B, _H // _HB),
        in_specs=[pl.BlockSpec((1, _HB, _DK, 128), lambda b, h: (b, h, 0, 0))],
        out_specs=qkv_spec,
        out_shape=jax.ShapeDtypeStruct((_B, _H, _DK, _L), jnp.float32),
        compiler_params=pltpu.CompilerParams(
            dimension_semantics=("parallel", "parallel"),
            vmem_limit_bytes=100 * 1024 * 1024),
    )(vm_all)

    return jnp.swapaxes(ctx_t, 2, 3), loss.reshape(())


# speedup vs baseline: 1.0319x; 1.0319x over previous
import math

import jax
import jax.numpy as jnp
from jax.experimental import pallas as pl
from jax.experimental.pallas import tpu as pltpu

_B, _H, _L, _DK = 4, 32, 4096, 64
_LK = 4096
_NC = 3
_LOG_L = 8
_LOG_LK = 8
_HB = 4


def _main_kernel(q_ref, k_ref, v_ref, wskT_ref, bsk_ref, wsvT_ref, bsv_ref,
                 wsqT_ref, bsq_ref, wpkT_ref, wpqT_ref,
                 bpq_ref, bpk_ref, wqp_ref, bqp_ref, wkp_ref, bkp_ref,
                 ctx_ref, loss_ref, zq_acc, zk_acc):
    b = pl.program_id(0)
    h = pl.program_id(1)

    @pl.when(jnp.logical_and(b == 0, h == 0))
    def _init():
        zq_acc[...] = jnp.zeros_like(zq_acc)
        zk_acc[...] = jnp.zeros_like(zk_acc)

    zpad = jnp.zeros((1, 128 - _NC), jnp.float32)
    cq_parts = []
    ck_parts = []
    for hh in range(_HB):
        q = q_ref[0, hh]
        k = k_ref[0, hh]
        v = v_ref[0, hh]

        ks = jnp.dot(k, wskT_ref[...], preferred_element_type=jnp.float32) + bsk_ref[...]
        qs = jnp.dot(q, wsqT_ref[...], preferred_element_type=jnp.float32) + bsq_ref[...]
        vs = jnp.dot(v, wsvT_ref[...], preferred_element_type=jnp.float32) + bsv_ref[...]

        vm = jnp.mean(vs, axis=1, keepdims=True)
        ctx_ref[0, hh] = jnp.broadcast_to(vm, (_DK, _L))

        ck_parts.append(jnp.concatenate(
            [jnp.sum(ks * wpkT_ref[c, hh], keepdims=True) for c in range(_NC)]
            + [zpad], axis=1))
        cq_parts.append(jnp.concatenate(
            [jnp.sum(qs * wpqT_ref[c, hh], keepdims=True) for c in range(_NC)]
            + [zpad], axis=1))

    onehot = (jax.lax.broadcasted_iota(jnp.int32, (_B, 1), 0) == b
              ).astype(jnp.float32)
    zq_acc[...] += onehot * sum(cq_parts)
    zk_acc[...] += onehot * sum(ck_parts)

    @pl.when(jnp.logical_and(b == _B - 1, h == _H // _HB - 1))
    def _finish():
        zq = zq_acc[...][:, :_NC]
        zk = zk_acc[...][:, :_NC]
        cqp = jnp.maximum(zq + bpq_ref[...], 0.0)
        ckp = jnp.maximum(zk + bpk_ref[...], 0.0)
        logit_q = jnp.dot(cqp, wqp_ref[...].T,
                          preferred_element_type=jnp.float32) + bqp_ref[...]
        logit_k = jnp.dot(ckp, wkp_ref[...].T,
                          preferred_element_type=jnp.float32) + bkp_ref[...]
        cluster_q = jax.nn.softmax(logit_q, axis=-1)
        cluster_k = jax.nn.softmax(logit_k, axis=-1)
        mu = jnp.mean(cluster_q, axis=0, keepdims=True)
        mk = jnp.mean(cluster_k, axis=0, keepdims=True)
        var = jnp.sum((cluster_k - mk) ** 2, axis=0, keepdims=True) / (_B - 1)
        sigma = jax.nn.softplus(jnp.sqrt(var))
        ll = (-0.5 * ((cluster_k - mu) / sigma) ** 2 - jnp.log(sigma)
              - 0.5 * math.log(2.0 * math.pi))
        lsm = jax.nn.log_softmax(cluster_q, axis=-1)
        ce_terms = jnp.sum(-cluster_q * lsm, axis=-1, keepdims=True)
        loss_ref[...] = (jnp.mean(ce_terms, axis=0, keepdims=True)
                         - jnp.mean(ll, keepdims=True))


def kernel(Q, K, V, W_sk, b_sk, W_sv, b_sv, W_sq, b_sq, W_pk, b_pk,
           W_pq, b_pq, W_qp, b_qp, W_kp, b_kp):
    Qt = jnp.swapaxes(Q, 2, 3)
    Kt = jnp.swapaxes(K, 2, 3)
    Vt = jnp.swapaxes(V, 2, 3)

    qkv_spec = pl.BlockSpec((1, _HB, _DK, _L), lambda b, h: (b, h, 0, 0))
    w_spec = pl.BlockSpec((_LK, _LOG_LK), lambda b, h: (0, 0))
    bias_spec = pl.BlockSpec((1, _LOG_LK), lambda b, h: (0, 0))
    wp_spec = pl.BlockSpec((_NC, _HB, _DK, _LOG_LK), lambda b, h: (0, h, 0, 0))
    small_spec = lambda r, c: pl.BlockSpec((r, c), lambda b, h: (0, 0))

    ctx_t, loss = pl.pallas_call(
        _main_kernel,
        grid=(_B, _H // _HB),
        in_specs=[qkv_spec, qkv_spec, qkv_spec,
                  w_spec, bias_spec, w_spec, bias_spec, w_spec, bias_spec,
                  wp_spec, wp_spec,
                  small_spec(1, _NC), small_spec(1, _NC),
                  small_spec(_NC, _NC), small_spec(1, _NC),
                  small_spec(_NC, _NC), small_spec(1, _NC)],
        out_specs=[qkv_spec, pl.BlockSpec((1, 1), lambda b, h: (0, 0))],
        out_shape=[
            jax.ShapeDtypeStruct((_B, _H, _DK, _L), jnp.float32),
            jax.ShapeDtypeStruct((1, 1), jnp.float32),
        ],
        scratch_shapes=[pltpu.VMEM((_B, 128), jnp.float32),
                        pltpu.VMEM((_B, 128), jnp.float32)],
        compiler_params=pltpu.CompilerParams(
            dimension_semantics=("arbitrary", "arbitrary"),
            vmem_limit_bytes=100 * 1024 * 1024),
    )(Qt, Kt, Vt,
      W_sk.T, b_sk.reshape(1, _LOG_LK), W_sv.T, b_sv.reshape(1, _LOG_LK),
      W_sq.T, b_sq.reshape(1, _LOG_L),
      W_pk.reshape(_NC, _H, _LOG_LK, _DK).transpose(0, 1, 3, 2),
      W_pq.reshape(_NC, _H, _LOG_L, _DK).transpose(0, 1, 3, 2),
      b_pq.reshape(1, _NC), b_pk.reshape(1, _NC),
      W_qp, b_qp.reshape(1, _NC), W_kp, b_kp.reshape(1, _NC))

    return jnp.swapaxes(ctx_t, 2, 3), loss.reshape(())
